# baseline (speedup 1.0000x reference)
import functools

import jax
import jax.numpy as jnp
from jax import lax
from jax.experimental import pallas as pl
from jax.experimental.pallas import tpu as pltpu
from jax.experimental.pallas import tpu_sc as plsc


def _precompute_body(tok_ref, pos_ref, w_ref, b_ref, e_ref, p_ref):
    dn = (((1,), (1,)), ((), ()))
    e_ref[...] = lax.dot_general(tok_ref[...], w_ref[...], dn,
                                 preferred_element_type=jnp.float32)
    p_ref[...] = lax.dot_general(pos_ref[...], w_ref[...], dn,
                                 preferred_element_type=jnp.float32) + b_ref[...]


def _precompute(tok_table, pos_t, W, b2d):
    V, _ = W.shape
    T = pos_t.shape[0]
    return pl.pallas_call(
        _precompute_body,
        out_shape=[
            jax.ShapeDtypeStruct((V, V), jnp.float32),
            jax.ShapeDtypeStruct((T, V), jnp.float32),
        ],
    )(tok_table, pos_t, W, b2d)


def _make_gather(V, T, BT):
    info = plsc.get_sparse_core_info()
    NC, NS = info.num_cores, info.num_subcores
    NW = NC * NS
    tok_per_w = BT // NW
    n_chunks = tok_per_w // T
    CPAD = 56
    n_lane_chunks = V // 16
    tail = V - 16

    mesh = plsc.VectorSubcoreMesh(core_axis_name="c", subcore_axis_name="s")

    @functools.partial(
        pl.kernel,
        mesh=mesh,
        out_type=jax.ShapeDtypeStruct((BT, V), jnp.float32),
        scratch_types=[
            pltpu.VMEM((n_chunks, CPAD), jnp.int32),
            pltpu.VMEM((CPAD, V), jnp.float32),
            pltpu.VMEM((T, V), jnp.float32),
            pltpu.VMEM((T, 16), jnp.float32),
            pltpu.SemaphoreType.DMA,
        ],
        compiler_params=pltpu.CompilerParams(use_tc_tiling_on_sc=False),
    )
    def gather_kernel(idx_hbm, e_hbm, p_hbm, pt_hbm, out_hbm,
                      idx_v, rows_v, p_v, pt_v, sem):
        wid = lax.axis_index("s") * NC + lax.axis_index("c")
        base = wid * n_chunks
        pltpu.sync_copy(idx_hbm.at[pl.ds(base, n_chunks), :], idx_v)
        pltpu.sync_copy(p_hbm, p_v)
        pltpu.sync_copy(pt_hbm, pt_v)

        def chunk_body(c, carry):
            pltpu.async_copy(
                e_hbm.at[idx_v.at[c, :]], rows_v, sem).wait()

            def row_body(i, carry2):
                for j in range(n_lane_chunks):
                    s = pl.ds(j * 16, 16)
                    rows_v[i, s] = rows_v[i, s] + p_v[i, s]
                s = pl.ds(tail, 16)
                rows_v[i, s] = rows_v[i, s] + pt_v[i, :]
                return carry2

            lax.fori_loop(0, T, row_body, 0)
            pltpu.sync_copy(
                rows_v.at[pl.ds(0, T), :],
                out_hbm.at[pl.ds((base + c) * T, T), :])
            return carry

        lax.fori_loop(0, n_chunks, chunk_body, 0)

    return gather_kernel


def kernel(indices, tok_table, pos_table, W, b):
    Bsz, T = indices.shape
    V = W.shape[0]
    BT = Bsz * T

    E, P = _precompute(tok_table, pos_table[:T], W, b.reshape(1, V))

    rem = V - 16 * (V // 16)
    pt = jnp.zeros((T, 16), jnp.float32)
    pt = pt.at[:, 16 - rem:].set(P[:, V - rem:])

    idx = indices.reshape(BT // T, T).astype(jnp.int32)
    idx_pad = jnp.zeros((BT // T, 56), jnp.int32).at[:, :T].set(idx)

    out = _make_gather(V, T, BT)(idx_pad, E, P, pt)
    return out.reshape(Bsz, T, V)
